# Initial kernel scaffold; baseline (speedup 1.0000x reference)
import jax
import jax.numpy as jnp
from jax.experimental import pallas as pl


def kernel(x, edge_index, batch, W0, b0, g0, be0, W1, b1, g1, be1, W2, b2, g2, be2, Wl, bl):
    raise NotImplementedError("write your pallas kernel here")


# speedup vs baseline: 5.6448x; 5.6448x over previous
import functools

import jax
import jax.numpy as jnp
from jax import lax
from jax.experimental import pallas as pl
from jax.experimental.pallas import tpu as pltpu
from jax.experimental.pallas import tpu_sc as plsc

N = 10000
D = 128
B = 64
C = 4
NC = 2
NS = 16
NW = NC * NS
K = 128

NP = 10240
E_PAD = NW * 80 * K
NB_PAD = NW * 3 * K
BP = 96
EPS = 1e-5


def _scatter_sum_sc(n_chunks, acc_rows, d, gather):
    units = acc_rows // 8
    per = -(-units // NS)
    assert units % per == 0
    n_active = units // per
    zr = per * 8
    mesh = plsc.VectorSubcoreMesh(core_axis_name="c", subcore_axis_name="s")

    scratch = [
        pltpu.VMEM_SHARED((acc_rows, d), jnp.float32),
        pltpu.VMEM((n_chunks, K), jnp.int32),
        pltpu.VMEM((K, d), jnp.float32),
        pltpu.SemaphoreType.DMA,
    ]
    if gather:
        scratch.append(pltpu.VMEM((n_chunks, K), jnp.int32))

    def body(*refs):
        if gather:
            table, src, dst, out, acc, idxd, rows, sem, idxs = refs
        else:
            dst, out, acc, idxd, rows, sem = refs
        cid = lax.axis_index("c")
        sid = lax.axis_index("s")
        wid = cid * NS + sid

        fill = jnp.zeros((16,), jnp.float32)

        def zrow(r, _):
            for c16 in range(d // 16):
                rows[r, pl.ds(c16 * 16, 16)] = fill
            return 0

        lax.fori_loop(0, K, zrow, 0)

        base_r = sid * zr
        nfull, rem = zr // K, zr % K

        @pl.when(sid < n_active)
        def _():
            for bchunk in range(nfull):
                pltpu.sync_copy(rows, acc.at[pl.ds(base_r + bchunk * K, K)])
            if rem:
                pltpu.sync_copy(rows.at[pl.ds(0, rem)],
                                acc.at[pl.ds(base_r + nfull * K, rem)])

        pltpu.sync_copy(dst.at[wid], idxd)
        if gather:
            pltpu.sync_copy(src.at[wid], idxs)

        if not gather:
            one = jnp.ones((16,), jnp.float32)

            def orow(r, _):
                for c16 in range(d // 16):
                    rows[r, pl.ds(c16 * 16, 16)] = one
                return 0

            lax.fori_loop(0, K, orow, 0)

        plsc.subcore_barrier()

        def chunk(j, _):
            if gather:
                pltpu.async_copy(table.at[idxs.at[j]], rows, sem).wait()
            pltpu.sync_copy(rows, acc.at[idxd.at[j]], add=True)
            return 0

        lax.fori_loop(0, n_chunks, chunk, 0)

        plsc.subcore_barrier()

        @pl.when(sid < n_active)
        def _():
            pltpu.sync_copy(acc.at[pl.ds(base_r, zr)],
                            out.at[cid, pl.ds(base_r, zr)])

    return functools.partial(
        pl.kernel,
        out_type=jax.ShapeDtypeStruct((NC, acc_rows, d), jnp.float32),
        mesh=mesh,
        scratch_types=scratch,
    )(body)


def _prep_tc(degP, x, w0):
    R = 1000

    def body(deg_ref, x_ref, w_ref, y_ref, dinv_ref):
        deg = deg_ref[0, :, 0:1] + deg_ref[1, :, 0:1] + 1.0
        dinv = lax.rsqrt(deg)
        y_ref[...] = jnp.dot(x_ref[...], w_ref[...],
                             preferred_element_type=jnp.float32) * dinv
        dinv_ref[...] = dinv

    return pl.pallas_call(
        body,
        grid=(N // R,),
        in_specs=[
            pl.BlockSpec((2, R, 16), lambda i: (0, i, 0)),
            pl.BlockSpec((R, D), lambda i: (i, 0)),
            pl.BlockSpec((D, D), lambda i: (0, 0)),
        ],
        out_specs=[
            pl.BlockSpec((R, D), lambda i: (i, 0)),
            pl.BlockSpec((R, 1), lambda i: (i, 0)),
        ],
        out_shape=[
            jax.ShapeDtypeStruct((N, D), jnp.float32),
            jax.ShapeDtypeStruct((N, 1), jnp.float32),
        ],
    )(degP, x, w0)


def _stats_tc(P, y, dinv, bvec):
    R = 1000
    nblk = N // R

    def body(p_ref, y_ref, dinv_ref, b_ref, out_ref, st_ref, acc):
        i = pl.program_id(0)
        o = (p_ref[0] + p_ref[1] + y_ref[...]) * dinv_ref[...] + b_ref[...]
        out_ref[...] = o

        @pl.when(i == 0)
        def _():
            acc[...] = jnp.zeros_like(acc)

        acc[0:1, :] = acc[0:1, :] + jnp.sum(o, axis=0, keepdims=True)
        acc[1:2, :] = acc[1:2, :] + jnp.sum(o * o, axis=0, keepdims=True)

        @pl.when(i == nblk - 1)
        def _():
            st_ref[...] = acc[...]

    return pl.pallas_call(
        body,
        grid=(nblk,),
        in_specs=[
            pl.BlockSpec((2, R, D), lambda i: (0, i, 0)),
            pl.BlockSpec((R, D), lambda i: (i, 0)),
            pl.BlockSpec((R, 1), lambda i: (i, 0)),
            pl.BlockSpec((1, D), lambda i: (0, 0)),
        ],
        out_specs=[
            pl.BlockSpec((R, D), lambda i: (i, 0)),
            pl.BlockSpec((8, D), lambda i: (0, 0)),
        ],
        out_shape=[
            jax.ShapeDtypeStruct((N, D), jnp.float32),
            jax.ShapeDtypeStruct((8, D), jnp.float32),
        ],
        scratch_shapes=[pltpu.VMEM((8, D), jnp.float32)],
    )(P, y, dinv, bvec)


def _next_tc(conv_out, stats, g, be, dinv, w):
    R = 1000

    def body(o_ref, st_ref, g_ref, be_ref, dinv_ref, w_ref, y_ref):
        m = st_ref[0:1, :] / N
        var = st_ref[1:2, :] / N - m * m
        a = g_ref[...] * lax.rsqrt(var + EPS)
        cv = be_ref[...] - m * a
        h = jnp.maximum(o_ref[...] * a + cv, 0.0)
        y_ref[...] = jnp.dot(h, w_ref[...],
                             preferred_element_type=jnp.float32) * dinv_ref[...]

    return pl.pallas_call(
        body,
        grid=(N // R,),
        in_specs=[
            pl.BlockSpec((R, D), lambda i: (i, 0)),
            pl.BlockSpec((8, D), lambda i: (0, 0)),
            pl.BlockSpec((1, D), lambda i: (0, 0)),
            pl.BlockSpec((1, D), lambda i: (0, 0)),
            pl.BlockSpec((R, 1), lambda i: (i, 0)),
            pl.BlockSpec((D, D), lambda i: (0, 0)),
        ],
        out_specs=pl.BlockSpec((R, D), lambda i: (i, 0)),
        out_shape=jax.ShapeDtypeStruct((N, D), jnp.float32),
    )(conv_out, stats, g, be, dinv, w)


def _act_tc(conv_out, stats, g, be):
    R = 1000

    def body(o_ref, st_ref, g_ref, be_ref, h_ref):
        m = st_ref[0:1, :] / N
        var = st_ref[1:2, :] / N - m * m
        a = g_ref[...] * lax.rsqrt(var + EPS)
        cv = be_ref[...] - m * a
        h_ref[...] = jnp.maximum(o_ref[...] * a + cv, 0.0)

    return pl.pallas_call(
        body,
        grid=(N // R,),
        in_specs=[
            pl.BlockSpec((R, D), lambda i: (i, 0)),
            pl.BlockSpec((8, D), lambda i: (0, 0)),
            pl.BlockSpec((1, D), lambda i: (0, 0)),
            pl.BlockSpec((1, D), lambda i: (0, 0)),
        ],
        out_specs=pl.BlockSpec((R, D), lambda i: (i, 0)),
        out_shape=jax.ShapeDtypeStruct((N, D), jnp.float32),
    )(conv_out, stats, g, be)


def _final_tc(poolP, cntP, wl, bl):
    def body(p_ref, c_ref, w_ref, b_ref, out_ref):
        s = p_ref[0, :B, :] + p_ref[1, :B, :]
        cnt = c_ref[0, :B, 0:1] + c_ref[1, :B, 0:1]
        pooled = s / jnp.maximum(cnt, 1.0)
        out_ref[...] = jnp.dot(pooled, w_ref[...],
                               preferred_element_type=jnp.float32) + b_ref[...]

    return pl.pallas_call(
        body,
        out_shape=jax.ShapeDtypeStruct((B, C), jnp.float32),
    )(poolP, cntP, wl, bl)


def kernel(x, edge_index, batch, W0, b0, g0, be0, W1, b1, g1, be1,
           W2, b2, g2, be2, Wl, bl):
    e = edge_index.shape[1]
    src = edge_index[0]
    dst = edge_index[1]
    src_p = jnp.concatenate(
        [src, jnp.zeros((E_PAD - e,), jnp.int32)]).reshape(NW, -1, K)
    dst_p = jnp.concatenate(
        [dst, jnp.full((E_PAD - e,), N, jnp.int32)]).reshape(NW, -1, K)
    batch_p = jnp.concatenate(
        [batch, jnp.full((NB_PAD - N,), B, jnp.int32)]).reshape(NW, -1, K)
    iota_p = jnp.concatenate([
        jnp.arange(N, dtype=jnp.int32),
        jnp.zeros((NB_PAD - N,), jnp.int32),
    ]).reshape(NW, -1, K)

    degP = _scatter_sum_sc(80, NP, 16, gather=False)(dst_p)
    cntP = _scatter_sum_sc(3, BP, 16, gather=False)(batch_p)

    y, dinv = _prep_tc(degP, x, W0)

    layers = [(b0, g0, be0, W1), (b1, g1, be1, W2), (b2, g2, be2, None)]
    h = None
    for bvec, g, be, w_next in layers:
        P = _scatter_sum_sc(80, NP, D, gather=True)(y, src_p, dst_p)
        conv_out, st = _stats_tc(P, y, dinv, bvec.reshape(1, D))
        if w_next is not None:
            y = _next_tc(conv_out, st, g.reshape(1, D), be.reshape(1, D),
                         dinv, w_next)
        else:
            h = _act_tc(conv_out, st, g.reshape(1, D), be.reshape(1, D))

    poolP = _scatter_sum_sc(3, BP, D, gather=True)(h, iota_p, batch_p)
    return _final_tc(poolP, cntP, Wl, bl.reshape(1, C))


# speedup vs baseline: 6.0608x; 1.0737x over previous
import functools

import jax
import jax.numpy as jnp
from jax import lax
from jax.experimental import pallas as pl
from jax.experimental.pallas import tpu as pltpu
from jax.experimental.pallas import tpu_sc as plsc

N = 10000
D = 128
B = 64
C = 4
NC = 2
NS = 16
NW = NC * NS
K = 128

NP = 10240
E_PAD = NW * 80 * K
NB_PAD = NW * 3 * K
BP = 96
EPS = 1e-5


def _scatter_sum_sc(n_chunks, acc_rows, d, gather):
    units = acc_rows // 8
    per = -(-units // NS)
    assert units % per == 0
    n_active = units // per
    zr = per * 8
    mesh = plsc.VectorSubcoreMesh(core_axis_name="c", subcore_axis_name="s")

    scratch = [
        pltpu.VMEM_SHARED((acc_rows, d), jnp.float32),
        pltpu.VMEM((n_chunks, K), jnp.int32),
        pltpu.VMEM((K, d), jnp.float32),
        pltpu.SemaphoreType.DMA,
    ]
    if gather:
        scratch += [
            pltpu.VMEM((1, K), jnp.int32),
            pltpu.VMEM((1, K), jnp.int32),
            pltpu.VMEM((K, d), jnp.float32),
            pltpu.SemaphoreType.DMA,
            pltpu.SemaphoreType.DMA,
            pltpu.SemaphoreType.DMA,
            pltpu.SemaphoreType.DMA,
        ]

    def body(*refs):
        if gather:
            (table, src, dst, out, acc, idxd, rows, sem, ibuf0, ibuf1,
             rows1, gsem1, ssem, isem0, isem1) = refs
        else:
            dst, out, acc, idxd, rows, sem = refs
        cid = lax.axis_index("c")
        sid = lax.axis_index("s")
        wid = cid * NS + sid

        fill = jnp.zeros((16,), jnp.float32)

        def zrow(r, _):
            for c16 in range(d // 16):
                rows[r, pl.ds(c16 * 16, 16)] = fill
            return 0

        lax.fori_loop(0, K, zrow, 0)

        base_r = sid * zr
        nfull, rem = zr // K, zr % K

        @pl.when(sid < n_active)
        def _():
            for bchunk in range(nfull):
                pltpu.sync_copy(rows, acc.at[pl.ds(base_r + bchunk * K, K)])
            if rem:
                pltpu.sync_copy(rows.at[pl.ds(0, rem)],
                                acc.at[pl.ds(base_r + nfull * K, rem)])

        pltpu.sync_copy(dst.at[wid], idxd)

        if not gather:
            one = jnp.ones((16,), jnp.float32)

            def orow(r, _):
                for c16 in range(d // 16):
                    rows[r, pl.ds(c16 * 16, 16)] = one
                return 0

            lax.fori_loop(0, K, orow, 0)

        plsc.subcore_barrier()

        if not gather:
            def chunk(j, _):
                pltpu.sync_copy(rows, acc.at[idxd.at[j]], add=True)
                return 0

            lax.fori_loop(0, n_chunks, chunk, 0)
        else:
            nt = n_chunks // 2
            tail = n_chunks % 2 == 1

            def i_start(j, ib, s):
                pltpu.async_copy(src.at[wid, j], ib, s)

            def i_wait(j, ib, s):
                pltpu.make_async_copy(src.at[wid, j], ib, s).wait()

            def g_start(j, ib, buf, gs):
                pltpu.async_copy(table.at[ib.at[0]], buf, gs)

            def g_wait(j, ib, buf, gs):
                pltpu.make_async_copy(table.at[ib.at[0]], buf, gs).wait()

            def s_start(j, buf):
                pltpu.async_copy(buf, acc.at[idxd.at[j]], ssem, add=True)

            def s_wait(j, buf):
                pltpu.make_async_copy(buf, acc.at[idxd.at[j]], ssem).wait()

            i_start(0, ibuf0, isem0)
            if n_chunks > 1:
                i_start(1, ibuf1, isem1)
            i_wait(0, ibuf0, isem0)
            g_start(0, ibuf0, rows, sem)

            def pair(t, _):
                j0 = 2 * t
                j1 = j0 + 1
                g_wait(j0, ibuf0, rows, sem)
                if tail:
                    i_start(j0 + 2, ibuf0, isem0)
                else:
                    @pl.when(t < nt - 1)
                    def _():
                        i_start(j0 + 2, ibuf0, isem0)

                @pl.when(t > 0)
                def _():
                    s_wait(j0 - 1, rows1)

                s_start(j0, rows)
                i_wait(j1, ibuf1, isem1)
                g_start(j1, ibuf1, rows1, gsem1)
                g_wait(j1, ibuf1, rows1, gsem1)

                @pl.when(t < nt - 1)
                def _():
                    i_start(j1 + 2, ibuf1, isem1)

                s_wait(j0, rows)
                s_start(j1, rows1)
                if tail:
                    i_wait(j0 + 2, ibuf0, isem0)
                    g_start(j0 + 2, ibuf0, rows, sem)
                else:
                    @pl.when(t < nt - 1)
                    def _():
                        i_wait(j0 + 2, ibuf0, isem0)
                        g_start(j0 + 2, ibuf0, rows, sem)
                return 0

            lax.fori_loop(0, nt, pair, 0)

            if tail:
                jt = n_chunks - 1
                g_wait(jt, ibuf0, rows, sem)
                if nt > 0:
                    s_wait(jt - 1, rows1)
                s_start(jt, rows)
                s_wait(jt, rows)
            else:
                s_wait(n_chunks - 1, rows1)

        plsc.subcore_barrier()

        @pl.when(sid < n_active)
        def _():
            pltpu.sync_copy(acc.at[pl.ds(base_r, zr)],
                            out.at[cid, pl.ds(base_r, zr)])

    return functools.partial(
        pl.kernel,
        out_type=jax.ShapeDtypeStruct((NC, acc_rows, d), jnp.float32),
        mesh=mesh,
        scratch_types=scratch,
    )(body)


def _prep_tc(degP, x, w0):
    R = 1000

    def body(deg_ref, x_ref, w_ref, y_ref, dinv_ref):
        deg = deg_ref[0, :, 0:1] + deg_ref[1, :, 0:1] + 1.0
        dinv = lax.rsqrt(deg)
        y_ref[...] = jnp.dot(x_ref[...], w_ref[...],
                             preferred_element_type=jnp.float32) * dinv
        dinv_ref[...] = dinv

    return pl.pallas_call(
        body,
        grid=(N // R,),
        in_specs=[
            pl.BlockSpec((2, R, 16), lambda i: (0, i, 0)),
            pl.BlockSpec((R, D), lambda i: (i, 0)),
            pl.BlockSpec((D, D), lambda i: (0, 0)),
        ],
        out_specs=[
            pl.BlockSpec((R, D), lambda i: (i, 0)),
            pl.BlockSpec((R, 1), lambda i: (i, 0)),
        ],
        out_shape=[
            jax.ShapeDtypeStruct((N, D), jnp.float32),
            jax.ShapeDtypeStruct((N, 1), jnp.float32),
        ],
    )(degP, x, w0)


def _stats_tc(P, y, dinv, bvec):
    R = 1000
    nblk = N // R

    def body(p_ref, y_ref, dinv_ref, b_ref, out_ref, st_ref, acc):
        i = pl.program_id(0)
        o = (p_ref[0] + p_ref[1] + y_ref[...]) * dinv_ref[...] + b_ref[...]
        out_ref[...] = o

        @pl.when(i == 0)
        def _():
            acc[...] = jnp.zeros_like(acc)

        acc[0:1, :] = acc[0:1, :] + jnp.sum(o, axis=0, keepdims=True)
        acc[1:2, :] = acc[1:2, :] + jnp.sum(o * o, axis=0, keepdims=True)

        @pl.when(i == nblk - 1)
        def _():
            st_ref[...] = acc[...]

    return pl.pallas_call(
        body,
        grid=(nblk,),
        in_specs=[
            pl.BlockSpec((2, R, D), lambda i: (0, i, 0)),
            pl.BlockSpec((R, D), lambda i: (i, 0)),
            pl.BlockSpec((R, 1), lambda i: (i, 0)),
            pl.BlockSpec((1, D), lambda i: (0, 0)),
        ],
        out_specs=[
            pl.BlockSpec((R, D), lambda i: (i, 0)),
            pl.BlockSpec((8, D), lambda i: (0, 0)),
        ],
        out_shape=[
            jax.ShapeDtypeStruct((N, D), jnp.float32),
            jax.ShapeDtypeStruct((8, D), jnp.float32),
        ],
        scratch_shapes=[pltpu.VMEM((8, D), jnp.float32)],
    )(P, y, dinv, bvec)


def _next_tc(conv_out, stats, g, be, dinv, w):
    R = 1000

    def body(o_ref, st_ref, g_ref, be_ref, dinv_ref, w_ref, y_ref):
        m = st_ref[0:1, :] / N
        var = st_ref[1:2, :] / N - m * m
        a = g_ref[...] * lax.rsqrt(var + EPS)
        cv = be_ref[...] - m * a
        h = jnp.maximum(o_ref[...] * a + cv, 0.0)
        y_ref[...] = jnp.dot(h, w_ref[...],
                             preferred_element_type=jnp.float32) * dinv_ref[...]

    return pl.pallas_call(
        body,
        grid=(N // R,),
        in_specs=[
            pl.BlockSpec((R, D), lambda i: (i, 0)),
            pl.BlockSpec((8, D), lambda i: (0, 0)),
            pl.BlockSpec((1, D), lambda i: (0, 0)),
            pl.BlockSpec((1, D), lambda i: (0, 0)),
            pl.BlockSpec((R, 1), lambda i: (i, 0)),
            pl.BlockSpec((D, D), lambda i: (0, 0)),
        ],
        out_specs=pl.BlockSpec((R, D), lambda i: (i, 0)),
        out_shape=jax.ShapeDtypeStruct((N, D), jnp.float32),
    )(conv_out, stats, g, be, dinv, w)


def _act_tc(conv_out, stats, g, be):
    R = 1000

    def body(o_ref, st_ref, g_ref, be_ref, h_ref):
        m = st_ref[0:1, :] / N
        var = st_ref[1:2, :] / N - m * m
        a = g_ref[...] * lax.rsqrt(var + EPS)
        cv = be_ref[...] - m * a
        h_ref[...] = jnp.maximum(o_ref[...] * a + cv, 0.0)

    return pl.pallas_call(
        body,
        grid=(N // R,),
        in_specs=[
            pl.BlockSpec((R, D), lambda i: (i, 0)),
            pl.BlockSpec((8, D), lambda i: (0, 0)),
            pl.BlockSpec((1, D), lambda i: (0, 0)),
            pl.BlockSpec((1, D), lambda i: (0, 0)),
        ],
        out_specs=pl.BlockSpec((R, D), lambda i: (i, 0)),
        out_shape=jax.ShapeDtypeStruct((N, D), jnp.float32),
    )(conv_out, stats, g, be)


def _final_tc(poolP, cntP, wl, bl):
    def body(p_ref, c_ref, w_ref, b_ref, out_ref):
        s = p_ref[0, :B, :] + p_ref[1, :B, :]
        cnt = c_ref[0, :B, 0:1] + c_ref[1, :B, 0:1]
        pooled = s / jnp.maximum(cnt, 1.0)
        out_ref[...] = jnp.dot(pooled, w_ref[...],
                               preferred_element_type=jnp.float32) + b_ref[...]

    return pl.pallas_call(
        body,
        out_shape=jax.ShapeDtypeStruct((B, C), jnp.float32),
    )(poolP, cntP, wl, bl)


def kernel(x, edge_index, batch, W0, b0, g0, be0, W1, b1, g1, be1,
           W2, b2, g2, be2, Wl, bl):
    e = edge_index.shape[1]
    src = edge_index[0]
    dst = edge_index[1]
    src_p = jnp.concatenate(
        [src, jnp.zeros((E_PAD - e,), jnp.int32)]).reshape(NW, -1, 1, K)
    dst_p = jnp.concatenate(
        [dst, jnp.full((E_PAD - e,), N, jnp.int32)]).reshape(NW, -1, K)
    batch_p = jnp.concatenate(
        [batch, jnp.full((NB_PAD - N,), B, jnp.int32)]).reshape(NW, -1, K)
    iota_p = jnp.concatenate([
        jnp.arange(N, dtype=jnp.int32),
        jnp.zeros((NB_PAD - N,), jnp.int32),
    ]).reshape(NW, -1, 1, K)

    degP = _scatter_sum_sc(80, NP, 16, gather=False)(dst_p)
    cntP = _scatter_sum_sc(3, BP, 16, gather=False)(batch_p)

    y, dinv = _prep_tc(degP, x, W0)

    layers = [(b0, g0, be0, W1), (b1, g1, be1, W2), (b2, g2, be2, None)]
    h = None
    for bvec, g, be, w_next in layers:
        P = _scatter_sum_sc(80, NP, D, gather=True)(y, src_p, dst_p)
        conv_out, st = _stats_tc(P, y, dinv, bvec.reshape(1, D))
        if w_next is not None:
            y = _next_tc(conv_out, st, g.reshape(1, D), be.reshape(1, D),
                         dinv, w_next)
        else:
            h = _act_tc(conv_out, st, g.reshape(1, D), be.reshape(1, D))

    poolP = _scatter_sum_sc(3, BP, D, gather=True)(h, iota_p, batch_p)
    return _final_tc(poolP, cntP, Wl, bl.reshape(1, C))
